# baseline (speedup 1.0000x reference)
import functools
import math

import jax
import jax.numpy as jnp
from jax import lax
from jax.experimental import pallas as pl
from jax.experimental.pallas import tpu as pltpu

_B = 1024
_D = 64
_M = 100000
_TEMP = 0.05
_TILE_M = 2000
_HALF = _TILE_M // 2
_STEPS = _M // _TILE_M
_B2 = 2 * _B
_SCALE = math.log2(math.e) / _TEMP
_LN2 = math.log(2.0)


def _main_body(x_ref, f_ref, mz_rgb_ref, mz_ir_ref, xn_ref, acc_ref):
    i = pl.program_id(0)

    @pl.when(i == 0)
    def _init():
        x = x_ref[...]
        n = jnp.sqrt(jnp.sum(x * x, axis=1, keepdims=True))
        xn_ref[...] = x * (_SCALE / jnp.maximum(n, 1e-12))
        acc_ref[...] = jnp.zeros_like(acc_ref)

    xn = xn_ref[...]
    la = jax.lax.dot_general(
        f_ref[:_HALF, :], xn, (((1,), (1,)), ((), ())),
        preferred_element_type=jnp.float32)
    lb = jax.lax.dot_general(
        f_ref[_HALF:, :], xn, (((1,), (1,)), ((), ())),
        preferred_element_type=jnp.float32)
    ones = jnp.ones((1, _HALF), jnp.float32)
    sa = jax.lax.dot_general(
        ones, jnp.exp2(la), (((1,), (0,)), ((), ())),
        preferred_element_type=jnp.float32)
    sb = jax.lax.dot_general(
        ones, jnp.exp2(lb), (((1,), (0,)), ((), ())),
        preferred_element_type=jnp.float32)
    acc_ref[...] += sa + sb

    @pl.when(i == _STEPS - 1)
    def _fini():
        logz = jnp.log(acc_ref[...])
        mz_rgb_ref[...] = jnp.mean(logz[:, :_B], axis=1, keepdims=True)
        mz_ir_ref[...] = jnp.mean(logz[:, _B:], axis=1, keepdims=True)


def _join_body(x_ref, g_ref, mz_rgb_ref, mz_ir_ref, out_rgb_ref, out_ir_ref):
    x = x_ref[...]
    n = jnp.sqrt(jnp.sum(x * x, axis=1, keepdims=True))
    xn = x / jnp.maximum(n, 1e-12)
    ll = jnp.sum(xn * g_ref[...], axis=1, keepdims=True) * (1.0 / _TEMP)
    out_rgb_ref[...] = mz_rgb_ref[...] - jnp.mean(ll[:_B, :], axis=0,
                                                  keepdims=True)
    out_ir_ref[...] = mz_ir_ref[...] - jnp.mean(ll[_B:, :], axis=0,
                                                keepdims=True)


def kernel(inputs_rgb, inputs_ir, targets_rgb, targets_ir, features):
    x = jnp.concatenate([inputs_rgb, inputs_ir], axis=0)
    t = jnp.concatenate([targets_rgb, targets_ir], axis=0).astype(jnp.int32)
    g = jnp.take(features, t, axis=0)
    mz_rgb, mz_ir = pl.pallas_call(
        _main_body,
        grid=(_STEPS,),
        in_specs=[
            pl.BlockSpec((_B2, _D), lambda i: (0, 0)),
            pl.BlockSpec((_TILE_M, _D), lambda i: (i, 0)),
        ],
        out_specs=[
            pl.BlockSpec((1, 1), lambda i: (0, 0)),
            pl.BlockSpec((1, 1), lambda i: (0, 0)),
        ],
        out_shape=[
            jax.ShapeDtypeStruct((1, 1), jnp.float32),
            jax.ShapeDtypeStruct((1, 1), jnp.float32),
        ],
        scratch_shapes=[
            pltpu.VMEM((_B2, _D), jnp.float32),
            pltpu.VMEM((1, _B2), jnp.float32),
        ],
        compiler_params=pltpu.CompilerParams(
            dimension_semantics=("arbitrary",)),
    )(x, features)
    out_rgb, out_ir = pl.pallas_call(
        _join_body,
        out_shape=[
            jax.ShapeDtypeStruct((1, 1), jnp.float32),
            jax.ShapeDtypeStruct((1, 1), jnp.float32),
        ],
    )(x, g, mz_rgb, mz_ir)
    return (out_rgb[0, 0], out_ir[0, 0])


# speedup vs baseline: 1.3127x; 1.3127x over previous
import functools
import math

import jax
import jax.numpy as jnp
from jax import lax
from jax.experimental import pallas as pl
from jax.experimental.pallas import tpu as pltpu

_B = 1024
_D = 64
_M = 100000
_TEMP = 0.05
_TILE_M = 4000
_HALF = _TILE_M // 2
_STEPS = _M // _TILE_M
_B2 = 2 * _B
_SCALE = math.log2(math.e) / _TEMP
_LN2 = math.log(2.0)


def _main_body(x_ref, f_ref, mz_rgb_ref, mz_ir_ref, xn_ref, acc_ref):
    i = pl.program_id(0)

    @pl.when(i == 0)
    def _init():
        x = x_ref[...]
        n = jnp.sqrt(jnp.sum(x * x, axis=1, keepdims=True))
        xn_ref[...] = x * (_SCALE / jnp.maximum(n, 1e-12))
        acc_ref[...] = jnp.zeros_like(acc_ref)

    xn = xn_ref[...]
    la = jax.lax.dot_general(
        f_ref[:_HALF, :], xn, (((1,), (1,)), ((), ())),
        preferred_element_type=jnp.float32)
    lb = jax.lax.dot_general(
        f_ref[_HALF:, :], xn, (((1,), (1,)), ((), ())),
        preferred_element_type=jnp.float32)
    s = (jnp.sum(jnp.exp2(la), axis=0, keepdims=True)
         + jnp.sum(jnp.exp2(lb), axis=0, keepdims=True))
    acc_ref[...] += s

    @pl.when(i == _STEPS - 1)
    def _fini():
        logz = jnp.log(acc_ref[...])
        mz_rgb_ref[...] = jnp.mean(logz[:, :_B], axis=1, keepdims=True)
        mz_ir_ref[...] = jnp.mean(logz[:, _B:], axis=1, keepdims=True)


def _join_body(x_ref, g_ref, mz_rgb_ref, mz_ir_ref, out_rgb_ref, out_ir_ref):
    x = x_ref[...]
    n = jnp.sqrt(jnp.sum(x * x, axis=1, keepdims=True))
    xn = x / jnp.maximum(n, 1e-12)
    ll = jnp.sum(xn * g_ref[...], axis=1, keepdims=True) * (1.0 / _TEMP)
    out_rgb_ref[...] = mz_rgb_ref[...] - jnp.mean(ll[:_B, :], axis=0,
                                                  keepdims=True)
    out_ir_ref[...] = mz_ir_ref[...] - jnp.mean(ll[_B:, :], axis=0,
                                                keepdims=True)


def kernel(inputs_rgb, inputs_ir, targets_rgb, targets_ir, features):
    x = jnp.concatenate([inputs_rgb, inputs_ir], axis=0)
    t = jnp.concatenate([targets_rgb, targets_ir], axis=0).astype(jnp.int32)
    g = jnp.take(features, t, axis=0)
    mz_rgb, mz_ir = pl.pallas_call(
        _main_body,
        grid=(_STEPS,),
        in_specs=[
            pl.BlockSpec((_B2, _D), lambda i: (0, 0)),
            pl.BlockSpec((_TILE_M, _D), lambda i: (i, 0)),
        ],
        out_specs=[
            pl.BlockSpec((1, 1), lambda i: (0, 0)),
            pl.BlockSpec((1, 1), lambda i: (0, 0)),
        ],
        out_shape=[
            jax.ShapeDtypeStruct((1, 1), jnp.float32),
            jax.ShapeDtypeStruct((1, 1), jnp.float32),
        ],
        scratch_shapes=[
            pltpu.VMEM((_B2, _D), jnp.float32),
            pltpu.VMEM((1, _B2), jnp.float32),
        ],
        compiler_params=pltpu.CompilerParams(
            dimension_semantics=("arbitrary",)),
    )(x, features)
    out_rgb, out_ir = pl.pallas_call(
        _join_body,
        out_shape=[
            jax.ShapeDtypeStruct((1, 1), jnp.float32),
            jax.ShapeDtypeStruct((1, 1), jnp.float32),
        ],
    )(x, g, mz_rgb, mz_ir)
    return (out_rgb[0, 0], out_ir[0, 0])
